# Initial kernel scaffold; baseline (speedup 1.0000x reference)
import jax
import jax.numpy as jnp
from jax.experimental import pallas as pl


def kernel(x, core_W, thresholds, axon_idx, out_idx):
    raise NotImplementedError("write your pallas kernel here")


# speedup vs baseline: 1.0827x; 1.0827x over previous
import functools

import jax
import jax.numpy as jnp
from jax.experimental import pallas as pl
from jax.experimental.pallas import tpu as pltpu

_T = 16


def _spike_words():
    xs = jnp.arange(17, dtype=jnp.float32) / 16.0
    word = jnp.zeros((17,), dtype=jnp.int32)
    for cycle in range(_T):
        N = jnp.round(xs * _T).astype(jnp.int32)
        mask = (N != 0) & (N != _T) & (cycle < _T)
        N_safe = jnp.maximum(N, 1)
        spacing = _T / N_safe.astype(jnp.float32)
        res = (
            mask
            & (jnp.floor(cycle / spacing) < N_safe.astype(jnp.float32))
            & (jnp.floor(jnp.mod(float(cycle), spacing)) == 0)
        )
        res = res.astype(jnp.float32)
        res = jnp.where(N == _T, 1.0, res)
        word = word | (res.astype(jnp.int32) << cycle)
    return word


def _body(xT_ref, wr_ref, thr_ref, adj_in_ref, adj_rec_ref, oidx_ref, word_ref,
          out_ref, pat_ref, prec_ref, chunk_ref, gpat_ref, memb_ref, fb_ref,
          fsum_ref):
    IN = xT_ref.shape[0]
    R = prec_ref.shape[0]
    F = fsum_ref.shape[0]
    B = xT_ref.shape[1]
    C = thr_ref.shape[0]
    CO = F // C
    CI = R // C
    ONN = out_ref.shape[0]

    f32 = jnp.float32

    n_int = jnp.round(xT_ref[...] * float(_T)).astype(jnp.int32)
    acc = jnp.zeros((IN, B), dtype=jnp.int32)
    for n in range(17):
        acc = jnp.where(n_int == n, word_ref[n], acc)
    pat_ref[...] = acc

    def _col(ref, r):
        a = ref[...]
        sel = jax.lax.broadcasted_iota(jnp.int32, a.shape, 1) == r
        return jnp.max(jnp.where(sel, a, -2), axis=1, keepdims=True)

    def in_gather(r, _):
        idx = _col(adj_in_ref, r)
        oh = (jax.lax.broadcasted_iota(jnp.int32, (512, IN), 1) == idx)
        chunk_ref[:, :IN] = oh.astype(f32)
        g = jax.lax.dot_general(
            chunk_ref[:, :IN], pat_ref[...].astype(f32),
            (((1,), (0,)), ((), ())),
            precision=jax.lax.Precision.HIGHEST,
            preferred_element_type=f32)
        gpat_ref[pl.ds(r * 512, 512), :] = g.astype(jnp.int32)
        return ()

    jax.lax.fori_loop(0, adj_in_ref.shape[1], in_gather, (), unroll=False)

    def gen_prec(r, _):
        idx = _col(adj_rec_ref, r)
        oh = (jax.lax.broadcasted_iota(jnp.int32, (512, F), 1) == idx)
        prec_ref[pl.ds(r * 512, 512), :] = oh.astype(jnp.bfloat16)
        return ()

    jax.lax.fori_loop(0, adj_rec_ref.shape[1], gen_prec, (), unroll=False)

    memb_ref[...] = jnp.zeros((F, B), f32)
    fsum_ref[...] = jnp.zeros((F, B), f32)
    fb_ref[0] = jnp.zeros((F, B), jnp.bfloat16)

    def cycle_body(t, _):
        cur = t % 2
        git = ((gpat_ref[...] >> t) & 1).astype(f32)
        fired_prev = fb_ref[cur]
        for c in range(C):
            rs = c * CI
            os_ = c * CO
            g_rec = jax.lax.dot_general(
                prec_ref[pl.ds(rs, CI), :], fired_prev,
                (((1,), (0,)), ((), ())),
                preferred_element_type=f32)
            in_sig = g_rec + git[rs:rs + CI, :]
            inc = jax.lax.dot_general(
                wr_ref[pl.ds(os_, CO), :], in_sig,
                (((1,), (0,)), ((), ())),
                preferred_element_type=f32)
            memb = memb_ref[pl.ds(os_, CO), :] + inc
            thr_c = thr_ref[c]
            fired = (thr_c < memb).astype(f32)
            memb_ref[pl.ds(os_, CO), :] = memb - fired * thr_c
            fb_ref[1 - cur, pl.ds(os_, CO), :] = fired.astype(jnp.bfloat16)
            fsum_ref[pl.ds(os_, CO), :] = fsum_ref[pl.ds(os_, CO), :] + fired
        return ()

    jax.lax.fori_loop(0, _T, cycle_body, (), unroll=False)

    def out_gather(r, _):
        idx = _col(oidx_ref, r)
        oh = (jax.lax.broadcasted_iota(jnp.int32, (512, F), 1) == idx)
        chunk_ref[...] = oh.astype(f32)
        out_ref[pl.ds(r * 512, 512), :] = jax.lax.dot_general(
            chunk_ref[...], fsum_ref[...],
            (((1,), (0,)), ((), ())),
            preferred_element_type=f32)
        return ()

    jax.lax.fori_loop(0, oidx_ref.shape[1], out_gather, (), unroll=False)


@jax.jit
def kernel(x, core_W, thresholds, axon_idx, out_idx):
    B, IN = x.shape
    C, COUT, CIN = core_W.shape
    R = C * CIN
    F = C * COUT
    ONN = out_idx.shape[0]

    word = _spike_words()
    xT = x.T
    wr = core_W.reshape(F, CIN)
    af = axon_idx.reshape(-1).astype(jnp.int32)
    adj_in = jnp.where(af < IN, af, -1).reshape(R // 512, 512).T
    adj_rec = jnp.where(af >= IN, af - IN, -1).reshape(R // 512, 512).T
    oidx = out_idx.astype(jnp.int32).reshape(ONN // 512, 512).T

    vm = pl.BlockSpec(memory_space=pltpu.VMEM)
    sm = pl.BlockSpec(memory_space=pltpu.SMEM)

    out_T = pl.pallas_call(
        _body,
        out_shape=jax.ShapeDtypeStruct((ONN, B), jnp.float32),
        in_specs=[vm, vm, sm, vm, vm, vm, sm],
        out_specs=vm,
        compiler_params=pltpu.CompilerParams(
            vmem_limit_bytes=100 * 1024 * 1024),
        scratch_shapes=[
            pltpu.VMEM((IN, B), jnp.int32),
            pltpu.VMEM((R, F), jnp.bfloat16),
            pltpu.VMEM((512, F), jnp.float32),
            pltpu.VMEM((R, B), jnp.int32),
            pltpu.VMEM((F, B), jnp.float32),
            pltpu.VMEM((2, F, B), jnp.bfloat16),
            pltpu.VMEM((F, B), jnp.float32),
        ],
    )(xT, wr, thresholds, adj_in, adj_rec, oidx, word)
    return out_T.T


# speedup vs baseline: 2.3975x; 2.2143x over previous
import jax
import jax.numpy as jnp
from jax.experimental import pallas as pl
from jax.experimental.pallas import tpu as pltpu

_T = 16


def _spike_words():
    xs = jnp.arange(17, dtype=jnp.float32) / 16.0
    word = jnp.zeros((17,), dtype=jnp.int32)
    for cycle in range(_T):
        N = jnp.round(xs * _T).astype(jnp.int32)
        mask = (N != 0) & (N != _T) & (cycle < _T)
        N_safe = jnp.maximum(N, 1)
        spacing = _T / N_safe.astype(jnp.float32)
        res = (
            mask
            & (jnp.floor(cycle / spacing) < N_safe.astype(jnp.float32))
            & (jnp.floor(jnp.mod(float(cycle), spacing)) == 0)
        )
        res = res.astype(jnp.float32)
        res = jnp.where(N == _T, 1.0, res)
        word = word | (res.astype(jnp.int32) << cycle)
    return word


def _body(x_ref, wrT_ref, thr_ref, adj_in_ref, adj_rec_ref, oidx_ref,
          word_ref, out_ref, pat_ref, precT_ref, chunk_ref, gpat_ref,
          memb_ref, fb_ref, fsum_ref):
    IN = x_ref.shape[1]
    F = fsum_ref.shape[1]
    B = x_ref.shape[0]
    C = thr_ref.shape[0]
    CW = F // C
    f32 = jnp.float32

    n_int = jnp.round(x_ref[...] * float(_T)).astype(jnp.int32)
    acc = jnp.zeros((B, IN), dtype=jnp.int32)
    for n in range(17):
        acc = jnp.where(n_int == n, word_ref[n], acc)
    pat_ref[...] = acc.astype(f32)

    for r in range(adj_in_ref.shape[0]):
        idx = adj_in_ref[r:r + 1, :]
        oh = (jax.lax.broadcasted_iota(jnp.int32, (IN, 512), 0) == idx)
        chunk_ref[:IN, :] = oh.astype(f32)
        g = jax.lax.dot_general(
            pat_ref[...], chunk_ref[:IN, :],
            (((1,), (0,)), ((), ())),
            precision=jax.lax.Precision.HIGHEST,
            preferred_element_type=f32)
        gpat_ref[:, r * 512:(r + 1) * 512] = g.astype(jnp.int32)

    for r in range(adj_rec_ref.shape[0]):
        idx = adj_rec_ref[r:r + 1, :]
        oh = (jax.lax.broadcasted_iota(jnp.int32, (F, 512), 0) == idx)
        precT_ref[:, r * 512:(r + 1) * 512] = oh.astype(jnp.bfloat16)

    memb_ref[...] = jnp.zeros((B, F), f32)
    fsum_ref[...] = jnp.zeros((B, F), f32)
    fb_ref[0] = jnp.zeros((B, F), jnp.bfloat16)

    def cycle_body(t, _):
        cur = t % 2
        git = ((gpat_ref[...] >> t) & 1).astype(f32)
        g_rec = jax.lax.dot_general(
            fb_ref[cur], precT_ref[...],
            (((1,), (0,)), ((), ())),
            preferred_element_type=f32)
        sig = g_rec + git
        for c in range(C):
            cs = c * CW
            inc = jax.lax.dot_general(
                sig[:, cs:cs + CW], wrT_ref[cs:cs + CW, :],
                (((1,), (0,)), ((), ())),
                preferred_element_type=f32)
            memb = memb_ref[:, cs:cs + CW] + inc
            thr_c = thr_ref[c]
            fired = (thr_c < memb).astype(f32)
            memb_ref[:, cs:cs + CW] = memb - fired * thr_c
            fb_ref[1 - cur, :, cs:cs + CW] = fired.astype(jnp.bfloat16)
            fsum_ref[:, cs:cs + CW] = fsum_ref[:, cs:cs + CW] + fired
        return ()

    jax.lax.fori_loop(0, _T, cycle_body, (), unroll=False)

    for r in range(oidx_ref.shape[0]):
        idx = oidx_ref[r:r + 1, :]
        oh = (jax.lax.broadcasted_iota(jnp.int32, (F, 512), 0) == idx)
        chunk_ref[...] = oh.astype(f32)
        out_ref[:, r * 512:(r + 1) * 512] = jax.lax.dot_general(
            fsum_ref[...], chunk_ref[...],
            (((1,), (0,)), ((), ())),
            preferred_element_type=f32)


@jax.jit
def kernel(x, core_W, thresholds, axon_idx, out_idx):
    B, IN = x.shape
    C, COUT, CIN = core_W.shape
    R = C * CIN
    F = C * COUT
    ONN = out_idx.shape[0]

    word = _spike_words()
    wrT = core_W.transpose(0, 2, 1).reshape(R, COUT)
    af = axon_idx.reshape(-1).astype(jnp.int32)
    adj_in = jnp.where(af < IN, af, -1).reshape(R // 512, 512)
    adj_rec = jnp.where(af >= IN, af - IN, -1).reshape(R // 512, 512)
    oidx = out_idx.astype(jnp.int32).reshape(ONN // 512, 512)

    vm = pl.BlockSpec(memory_space=pltpu.VMEM)
    sm = pl.BlockSpec(memory_space=pltpu.SMEM)

    return pl.pallas_call(
        _body,
        out_shape=jax.ShapeDtypeStruct((B, ONN), jnp.float32),
        in_specs=[vm, vm, sm, vm, vm, vm, sm],
        out_specs=vm,
        compiler_params=pltpu.CompilerParams(
            vmem_limit_bytes=100 * 1024 * 1024),
        scratch_shapes=[
            pltpu.VMEM((B, IN), jnp.float32),
            pltpu.VMEM((F, R), jnp.bfloat16),
            pltpu.VMEM((F, 512), jnp.float32),
            pltpu.VMEM((B, R), jnp.int32),
            pltpu.VMEM((B, F), jnp.float32),
            pltpu.VMEM((2, B, F), jnp.bfloat16),
            pltpu.VMEM((B, F), jnp.float32),
        ],
    )(x, wrT, thresholds, adj_in, adj_rec, oidx, word)


# speedup vs baseline: 2.4045x; 1.0029x over previous
import jax
import jax.numpy as jnp
from jax.experimental import pallas as pl
from jax.experimental.pallas import tpu as pltpu

_T = 16


def _spike_words():
    xs = jnp.arange(17, dtype=jnp.float32) / 16.0
    word = jnp.zeros((17,), dtype=jnp.int32)
    for cycle in range(_T):
        N = jnp.round(xs * _T).astype(jnp.int32)
        mask = (N != 0) & (N != _T) & (cycle < _T)
        N_safe = jnp.maximum(N, 1)
        spacing = _T / N_safe.astype(jnp.float32)
        res = (
            mask
            & (jnp.floor(cycle / spacing) < N_safe.astype(jnp.float32))
            & (jnp.floor(jnp.mod(float(cycle), spacing)) == 0)
        )
        res = res.astype(jnp.float32)
        res = jnp.where(N == _T, 1.0, res)
        word = word | (res.astype(jnp.int32) << cycle)
    return word


def _body(x_ref, wrT_ref, thr_ref, adj_in_ref, adj_rec_ref, oidx_ref,
          word_ref, out_ref, pat_ref, precT_ref, chunk_ref, gpat_ref,
          memb_ref, fb_ref, fsum_ref):
    IN = x_ref.shape[1]
    F = fsum_ref.shape[1]
    B = x_ref.shape[0]
    C = thr_ref.shape[0]
    CW = F // C
    f32 = jnp.float32

    n_int = jnp.round(x_ref[...] * float(_T)).astype(jnp.int32)
    acc = jnp.zeros((B, IN), dtype=jnp.int32)
    for n in range(17):
        acc = jnp.where(n_int == n, word_ref[n], acc)
    pat_ref[...] = acc.astype(f32)

    for r in range(adj_in_ref.shape[0]):
        idx = adj_in_ref[r:r + 1, :]
        oh = (jax.lax.broadcasted_iota(jnp.int32, (IN, 512), 0) == idx)
        chunk_ref[:IN, :] = oh.astype(f32)
        g = jax.lax.dot_general(
            pat_ref[...], chunk_ref[:IN, :],
            (((1,), (0,)), ((), ())),
            precision=jax.lax.Precision.HIGHEST,
            preferred_element_type=f32)
        gpat_ref[:, r * 512:(r + 1) * 512] = g.astype(jnp.int32)

    for r in range(adj_rec_ref.shape[0]):
        idx = adj_rec_ref[r:r + 1, :]
        oh = (jax.lax.broadcasted_iota(jnp.int32, (F, 512), 0) == idx)
        precT_ref[:, r * 512:(r + 1) * 512] = oh.astype(jnp.int8)

    memb_ref[...] = jnp.zeros((B, F), f32)
    fsum_ref[...] = jnp.zeros((B, F), f32)
    fb_ref[0] = jnp.zeros((B, F), jnp.int8)

    def cycle_body(t, _):
        cur = t % 2
        git = (gpat_ref[...] >> t) & 1
        g_rec = jax.lax.dot_general(
            fb_ref[cur], precT_ref[...],
            (((1,), (0,)), ((), ())),
            preferred_element_type=jnp.int32)
        sig = (g_rec + git).astype(f32)
        for c in range(C):
            cs = c * CW
            inc = jax.lax.dot_general(
                sig[:, cs:cs + CW], wrT_ref[cs:cs + CW, :],
                (((1,), (0,)), ((), ())),
                preferred_element_type=f32)
            memb = memb_ref[:, cs:cs + CW] + inc
            thr_c = thr_ref[c]
            fired = (thr_c < memb).astype(f32)
            memb_ref[:, cs:cs + CW] = memb - fired * thr_c
            fb_ref[1 - cur, :, cs:cs + CW] = fired.astype(jnp.int8)
            fsum_ref[:, cs:cs + CW] = fsum_ref[:, cs:cs + CW] + fired
        return ()

    jax.lax.fori_loop(0, _T, cycle_body, (), unroll=False)

    for r in range(oidx_ref.shape[0]):
        idx = oidx_ref[r:r + 1, :]
        oh = (jax.lax.broadcasted_iota(jnp.int32, (F, 512), 0) == idx)
        chunk_ref[...] = oh.astype(f32)
        out_ref[:, r * 512:(r + 1) * 512] = jax.lax.dot_general(
            fsum_ref[...], chunk_ref[...],
            (((1,), (0,)), ((), ())),
            preferred_element_type=f32)


@jax.jit
def kernel(x, core_W, thresholds, axon_idx, out_idx):
    B, IN = x.shape
    C, COUT, CIN = core_W.shape
    R = C * CIN
    F = C * COUT
    ONN = out_idx.shape[0]

    word = _spike_words()
    wrT = core_W.transpose(0, 2, 1).reshape(R, COUT)
    af = axon_idx.reshape(-1).astype(jnp.int32)
    adj_in = jnp.where(af < IN, af, -1).reshape(R // 512, 512)
    adj_rec = jnp.where(af >= IN, af - IN, -1).reshape(R // 512, 512)
    oidx = out_idx.astype(jnp.int32).reshape(ONN // 512, 512)

    vm = pl.BlockSpec(memory_space=pltpu.VMEM)
    sm = pl.BlockSpec(memory_space=pltpu.SMEM)

    return pl.pallas_call(
        _body,
        out_shape=jax.ShapeDtypeStruct((B, ONN), jnp.float32),
        in_specs=[vm, vm, sm, vm, vm, vm, sm],
        out_specs=vm,
        compiler_params=pltpu.CompilerParams(
            vmem_limit_bytes=100 * 1024 * 1024),
        scratch_shapes=[
            pltpu.VMEM((B, IN), jnp.float32),
            pltpu.VMEM((F, R), jnp.int8),
            pltpu.VMEM((F, 512), jnp.float32),
            pltpu.VMEM((B, R), jnp.int32),
            pltpu.VMEM((B, F), jnp.float32),
            pltpu.VMEM((2, B, F), jnp.int8),
            pltpu.VMEM((B, F), jnp.float32),
        ],
    )(x, wrT, thresholds, adj_in, adj_rec, oidx, word)


# speedup vs baseline: 3.6612x; 1.5226x over previous
import jax
import jax.numpy as jnp
from jax.experimental import pallas as pl
from jax.experimental.pallas import tpu as pltpu

_T = 16


def _spike_words():
    xs = jnp.arange(17, dtype=jnp.float32) / 16.0
    word = jnp.zeros((17,), dtype=jnp.int32)
    for cycle in range(_T):
        N = jnp.round(xs * _T).astype(jnp.int32)
        mask = (N != 0) & (N != _T) & (cycle < _T)
        N_safe = jnp.maximum(N, 1)
        spacing = _T / N_safe.astype(jnp.float32)
        res = (
            mask
            & (jnp.floor(cycle / spacing) < N_safe.astype(jnp.float32))
            & (jnp.floor(jnp.mod(float(cycle), spacing)) == 0)
        )
        res = res.astype(jnp.float32)
        res = jnp.where(N == _T, 1.0, res)
        word = word | (res.astype(jnp.int32) << cycle)
    return word


def _body(x_ref, wrT_ref, thr_ref, adj_in_ref, adj_rec4_ref, shift_ref,
          pk_ref, oidx_ref, word_ref, out_ref, pat_ref, precT_ref,
          chunk_ref, gpat_ref, memb_ref, fb_ref, fsum_ref):
    IN = x_ref.shape[1]
    F = fsum_ref.shape[1]
    B = x_ref.shape[0]
    C = thr_ref.shape[0]
    CW = F // C
    f32 = jnp.float32

    n_int = jnp.round(x_ref[...] * float(_T)).astype(jnp.int32)
    acc = jnp.zeros((B, IN), dtype=jnp.int32)
    for n in range(17):
        acc = jnp.where(n_int == n, word_ref[n], acc)
    pat_ref[...] = acc.astype(f32)

    for r in range(adj_in_ref.shape[0]):
        idx = adj_in_ref[r:r + 1, :]
        oh = (jax.lax.broadcasted_iota(jnp.int32, (IN, 512), 0) == idx)
        chunk_ref[:IN, :] = oh.astype(f32)
        g = jax.lax.dot_general(
            pat_ref[...], chunk_ref[:IN, :],
            (((1,), (0,)), ((), ())),
            precision=jax.lax.Precision.HIGHEST,
            preferred_element_type=f32)
        gpat_ref[:, r * 512:(r + 1) * 512] = g.astype(jnp.int32)

    F4 = F // 4
    for r in range(adj_rec4_ref.shape[0]):
        idx = adj_rec4_ref[r:r + 1, :]
        oh = (jax.lax.broadcasted_iota(jnp.int32, (F4, 512), 0) == idx)
        precT_ref[:, r * 512:(r + 1) * 512] = oh.astype(jnp.int8)

    memb_ref[...] = jnp.zeros((B, F), f32)
    fsum_ref[...] = jnp.zeros((B, F), f32)
    fb_ref[0] = jnp.zeros((B, F4), jnp.int8)

    def cycle_body(t, _):
        cur = t % 2
        git = (gpat_ref[...] >> t) & 1
        g_rec = jax.lax.dot_general(
            fb_ref[cur], precT_ref[...],
            (((1,), (0,)), ((), ())),
            preferred_element_type=jnp.int32)
        rec_bit = (g_rec >> shift_ref[...]) & 1
        sig = (rec_bit + git).astype(f32)
        for c in range(C):
            cs = c * CW
            inc = jax.lax.dot_general(
                sig[:, cs:cs + CW], wrT_ref[cs:cs + CW, :],
                (((1,), (0,)), ((), ())),
                preferred_element_type=f32)
            memb = memb_ref[:, cs:cs + CW] + inc
            thr_c = thr_ref[c]
            fired = (thr_c < memb).astype(f32)
            memb_ref[:, cs:cs + CW] = memb - fired * thr_c
            pf = jax.lax.dot_general(
                fired.astype(jnp.int8), pk_ref[...],
                (((1,), (0,)), ((), ())),
                preferred_element_type=jnp.int32)
            fb_ref[1 - cur, :, c * (CW // 4):(c + 1) * (CW // 4)] = (
                pf.astype(jnp.int8))
            fsum_ref[:, cs:cs + CW] = fsum_ref[:, cs:cs + CW] + fired
        return ()

    jax.lax.fori_loop(0, _T, cycle_body, (), unroll=False)

    for r in range(oidx_ref.shape[0]):
        idx = oidx_ref[r:r + 1, :]
        oh = (jax.lax.broadcasted_iota(jnp.int32, (F, 512), 0) == idx)
        chunk_ref[...] = oh.astype(f32)
        out_ref[:, r * 512:(r + 1) * 512] = jax.lax.dot_general(
            fsum_ref[...], chunk_ref[...],
            (((1,), (0,)), ((), ())),
            preferred_element_type=f32)


@jax.jit
def kernel(x, core_W, thresholds, axon_idx, out_idx):
    B, IN = x.shape
    C, COUT, CIN = core_W.shape
    R = C * CIN
    F = C * COUT
    ONN = out_idx.shape[0]

    word = _spike_words()
    wrT = core_W.transpose(0, 2, 1).reshape(R, COUT)
    af = axon_idx.reshape(-1).astype(jnp.int32)
    adj_in = jnp.where(af < IN, af, -1).reshape(R // 512, 512)
    adj_rec4 = jnp.where(af >= IN, (af - IN) // 4, -1).reshape(R // 512, 512)
    shift_row = jnp.where(af >= IN, 2 * ((af - IN) % 4), 0).reshape(1, R)
    i = jnp.arange(512)
    pk = ((i[:, None] // 4 == jnp.arange(128)[None, :]).astype(jnp.int32)
          * (1 << (2 * (i % 4)))[:, None]).astype(jnp.int8)
    oidx = out_idx.astype(jnp.int32).reshape(ONN // 512, 512)

    vm = pl.BlockSpec(memory_space=pltpu.VMEM)
    sm = pl.BlockSpec(memory_space=pltpu.SMEM)

    return pl.pallas_call(
        _body,
        out_shape=jax.ShapeDtypeStruct((B, ONN), jnp.float32),
        in_specs=[vm, vm, sm, vm, vm, vm, vm, vm, sm],
        out_specs=vm,
        compiler_params=pltpu.CompilerParams(
            vmem_limit_bytes=100 * 1024 * 1024),
        scratch_shapes=[
            pltpu.VMEM((B, IN), jnp.float32),
            pltpu.VMEM((F // 4, R), jnp.int8),
            pltpu.VMEM((F, 512), jnp.float32),
            pltpu.VMEM((B, R), jnp.int32),
            pltpu.VMEM((B, F), jnp.float32),
            pltpu.VMEM((2, B, F // 4), jnp.int8),
            pltpu.VMEM((B, F), jnp.float32),
        ],
    )(x, wrT, thresholds, adj_in, adj_rec4, shift_row, pk, oidx, word)


# speedup vs baseline: 4.1354x; 1.1295x over previous
import jax
import jax.numpy as jnp
from jax.experimental import pallas as pl
from jax.experimental.pallas import tpu as pltpu

_T = 16


def _spike_words():
    xs = jnp.arange(17, dtype=jnp.float32) / 16.0
    word = jnp.zeros((17,), dtype=jnp.int32)
    for cycle in range(_T):
        N = jnp.round(xs * _T).astype(jnp.int32)
        mask = (N != 0) & (N != _T) & (cycle < _T)
        N_safe = jnp.maximum(N, 1)
        spacing = _T / N_safe.astype(jnp.float32)
        res = (
            mask
            & (jnp.floor(cycle / spacing) < N_safe.astype(jnp.float32))
            & (jnp.floor(jnp.mod(float(cycle), spacing)) == 0)
        )
        res = res.astype(jnp.float32)
        res = jnp.where(N == _T, 1.0, res)
        word = word | (res.astype(jnp.int32) << cycle)
    return word


def _body(x_ref, wrT_ref, thr_ref, adj_in_ref, adj_rec4_ref, shift_ref,
          pk_ref, oidx_ref, word_ref, out_ref, pat_ref, precT_ref,
          chunk_ref, gpat_ref, memb_ref, fb_ref, fsum_ref):
    IN = x_ref.shape[1]
    F = fsum_ref.shape[1]
    B = x_ref.shape[0]
    C = thr_ref.shape[0]
    CW = F // C
    f32 = jnp.float32

    n_int = jnp.round(x_ref[...] * float(_T)).astype(jnp.int32)
    acc = jnp.zeros((B, IN), dtype=jnp.int32)
    for n in range(17):
        acc = jnp.where(n_int == n, word_ref[n], acc)
    pat_ref[0] = (acc & 255).astype(jnp.bfloat16)
    pat_ref[1] = (acc >> 8).astype(jnp.bfloat16)

    for r in range(adj_in_ref.shape[0]):
        idx = adj_in_ref[r:r + 1, :]
        oh = (jax.lax.broadcasted_iota(jnp.int32, (IN, 512), 0) == idx)
        chunk_ref[:IN, :] = oh.astype(jnp.bfloat16)
        glo = jax.lax.dot_general(
            pat_ref[0], chunk_ref[:IN, :],
            (((1,), (0,)), ((), ())),
            preferred_element_type=f32)
        ghi = jax.lax.dot_general(
            pat_ref[1], chunk_ref[:IN, :],
            (((1,), (0,)), ((), ())),
            preferred_element_type=f32)
        gpat_ref[:, r * 512:(r + 1) * 512] = (
            (ghi.astype(jnp.int32) << 8) | glo.astype(jnp.int32))

    F4 = F // 4
    for r in range(adj_rec4_ref.shape[0]):
        idx = adj_rec4_ref[r:r + 1, :]
        oh = (jax.lax.broadcasted_iota(jnp.int32, (F4, 512), 0) == idx)
        precT_ref[:, r * 512:(r + 1) * 512] = oh.astype(jnp.int8)

    memb_ref[...] = jnp.zeros((B, F), f32)
    fsum_ref[...] = jnp.zeros((B, F), f32)
    fb_ref[0] = jnp.zeros((B, F4), jnp.int8)

    def cycle_body(t, _):
        cur = t % 2
        git = (gpat_ref[...] >> t) & 1
        g_rec = jax.lax.dot_general(
            fb_ref[cur], precT_ref[...],
            (((1,), (0,)), ((), ())),
            preferred_element_type=jnp.int32)
        rec_bit = (g_rec >> shift_ref[...]) & 1
        sig = (rec_bit + git).astype(f32)
        for c in range(C):
            cs = c * CW
            inc = jax.lax.dot_general(
                sig[:, cs:cs + CW], wrT_ref[cs:cs + CW, :],
                (((1,), (0,)), ((), ())),
                preferred_element_type=f32)
            memb = memb_ref[:, cs:cs + CW] + inc
            thr_c = thr_ref[c]
            fired = (thr_c < memb).astype(f32)
            memb_ref[:, cs:cs + CW] = memb - fired * thr_c
            pf = jax.lax.dot_general(
                fired.astype(jnp.int8), pk_ref[...],
                (((1,), (0,)), ((), ())),
                preferred_element_type=jnp.int32)
            fb_ref[1 - cur, :, c * (CW // 4):(c + 1) * (CW // 4)] = (
                pf.astype(jnp.int8))
            fsum_ref[:, cs:cs + CW] = fsum_ref[:, cs:cs + CW] + fired
        return ()

    jax.lax.fori_loop(0, _T, cycle_body, (), unroll=False)

    for r in range(oidx_ref.shape[0]):
        idx = oidx_ref[r:r + 1, :]
        oh = (jax.lax.broadcasted_iota(jnp.int32, (F, 512), 0) == idx)
        chunk_ref[...] = oh.astype(jnp.bfloat16)
        out_ref[:, r * 512:(r + 1) * 512] = jax.lax.dot_general(
            fsum_ref[...].astype(jnp.bfloat16), chunk_ref[...],
            (((1,), (0,)), ((), ())),
            preferred_element_type=f32)


@jax.jit
def kernel(x, core_W, thresholds, axon_idx, out_idx):
    B, IN = x.shape
    C, COUT, CIN = core_W.shape
    R = C * CIN
    F = C * COUT
    ONN = out_idx.shape[0]

    word = _spike_words()
    wrT = core_W.transpose(0, 2, 1).reshape(R, COUT)
    af = axon_idx.reshape(-1).astype(jnp.int32)
    adj_in = jnp.where(af < IN, af, -1).reshape(R // 512, 512)
    adj_rec4 = jnp.where(af >= IN, (af - IN) // 4, -1).reshape(R // 512, 512)
    shift_row = jnp.where(af >= IN, 2 * ((af - IN) % 4), 0).reshape(1, R)
    i = jnp.arange(512)
    pk = ((i[:, None] // 4 == jnp.arange(128)[None, :]).astype(jnp.int32)
          * (1 << (2 * (i % 4)))[:, None]).astype(jnp.int8)
    oidx = out_idx.astype(jnp.int32).reshape(ONN // 512, 512)

    vm = pl.BlockSpec(memory_space=pltpu.VMEM)
    sm = pl.BlockSpec(memory_space=pltpu.SMEM)

    return pl.pallas_call(
        _body,
        out_shape=jax.ShapeDtypeStruct((B, ONN), jnp.float32),
        in_specs=[vm, vm, sm, vm, vm, vm, vm, vm, sm],
        out_specs=vm,
        compiler_params=pltpu.CompilerParams(
            vmem_limit_bytes=100 * 1024 * 1024),
        scratch_shapes=[
            pltpu.VMEM((2, B, IN), jnp.bfloat16),
            pltpu.VMEM((F // 4, R), jnp.int8),
            pltpu.VMEM((F, 512), jnp.bfloat16),
            pltpu.VMEM((B, R), jnp.int32),
            pltpu.VMEM((B, F), jnp.float32),
            pltpu.VMEM((2, B, F // 4), jnp.int8),
            pltpu.VMEM((B, F), jnp.float32),
        ],
    )(x, wrT, thresholds, adj_in, adj_rec4, shift_row, pk, oidx, word)


# speedup vs baseline: 4.8528x; 1.1735x over previous
import jax
import jax.numpy as jnp
from jax.experimental import pallas as pl
from jax.experimental.pallas import tpu as pltpu

_T = 16


def _spike_words():
    xs = jnp.arange(17, dtype=jnp.float32) / 16.0
    word = jnp.zeros((17,), dtype=jnp.int32)
    for cycle in range(_T):
        N = jnp.round(xs * _T).astype(jnp.int32)
        mask = (N != 0) & (N != _T) & (cycle < _T)
        N_safe = jnp.maximum(N, 1)
        spacing = _T / N_safe.astype(jnp.float32)
        res = (
            mask
            & (jnp.floor(cycle / spacing) < N_safe.astype(jnp.float32))
            & (jnp.floor(jnp.mod(float(cycle), spacing)) == 0)
        )
        res = res.astype(jnp.float32)
        res = jnp.where(N == _T, 1.0, res)
        word = word | (res.astype(jnp.int32) << cycle)
    return word


def _body(x_ref, wrT_ref, thr_ref, adj_in_ref, adj_rec4_ref, shift_ref,
          pk_ref, oidx_ref, word_ref, out_ref, pat_ref, precT_ref,
          chunk_ref, gpat_ref, memb_ref, fb_ref, fsum_ref):
    IN = x_ref.shape[1]
    F = fsum_ref.shape[1]
    B = x_ref.shape[0]
    C = thr_ref.shape[0]
    CW = F // C
    f32 = jnp.float32

    n_int = jnp.round(x_ref[...] * float(_T)).astype(jnp.int32)
    acc = jnp.zeros((B, IN), dtype=jnp.int32)
    for n in range(17):
        acc = jnp.where(n_int == n, word_ref[n], acc)
    pat_ref[0] = (acc & 255).astype(jnp.bfloat16)
    pat_ref[1] = (acc >> 8).astype(jnp.bfloat16)

    for r in range(adj_in_ref.shape[0]):
        idx = adj_in_ref[r:r + 1, :]
        oh = (jax.lax.broadcasted_iota(jnp.int32, (IN, 512), 0) == idx)
        chunk_ref[:IN, :] = oh.astype(jnp.bfloat16)
        glo = jax.lax.dot_general(
            pat_ref[0], chunk_ref[:IN, :],
            (((1,), (0,)), ((), ())),
            preferred_element_type=f32)
        ghi = jax.lax.dot_general(
            pat_ref[1], chunk_ref[:IN, :],
            (((1,), (0,)), ((), ())),
            preferred_element_type=f32)
        gpat_ref[:, r * 512:(r + 1) * 512] = (
            (ghi.astype(jnp.int32) << 8) | glo.astype(jnp.int32))

    F8 = F // 8
    for r in range(adj_rec4_ref.shape[0]):
        idx = adj_rec4_ref[r:r + 1, :]
        oh = (jax.lax.broadcasted_iota(jnp.int32, (F8, 512), 0) == idx)
        precT_ref[:, r * 512:(r + 1) * 512] = oh.astype(jnp.bfloat16)

    memb_ref[...] = jnp.zeros((B, F), f32)
    fsum_ref[...] = jnp.zeros((B, F), f32)
    fb_ref[0] = jnp.zeros((B, F8), jnp.bfloat16)

    def cycle_body(t, _):
        cur = t % 2
        git = (gpat_ref[...] >> t) & 1
        g_rec = jax.lax.dot_general(
            fb_ref[cur], precT_ref[...],
            (((1,), (0,)), ((), ())),
            preferred_element_type=f32)
        rec_bit = (g_rec.astype(jnp.int32) >> shift_ref[...]) & 1
        sig = (rec_bit + git).astype(f32)
        for c in range(C):
            cs = c * CW
            inc = jax.lax.dot_general(
                sig[:, cs:cs + CW], wrT_ref[cs:cs + CW, :],
                (((1,), (0,)), ((), ())),
                preferred_element_type=f32)
            memb = memb_ref[:, cs:cs + CW] + inc
            thr_c = thr_ref[c]
            fired = (thr_c < memb).astype(f32)
            memb_ref[:, cs:cs + CW] = memb - fired * thr_c
            pf = jax.lax.dot_general(
                fired.astype(jnp.bfloat16), pk_ref[...],
                (((1,), (0,)), ((), ())),
                preferred_element_type=f32)
            fb_ref[1 - cur, :, c * (CW // 8):(c + 1) * (CW // 8)] = (
                pf.astype(jnp.bfloat16))
            fsum_ref[:, cs:cs + CW] = fsum_ref[:, cs:cs + CW] + fired
        return ()

    jax.lax.fori_loop(0, _T, cycle_body, (), unroll=False)

    for r in range(oidx_ref.shape[0]):
        idx = oidx_ref[r:r + 1, :]
        oh = (jax.lax.broadcasted_iota(jnp.int32, (F, 512), 0) == idx)
        chunk_ref[...] = oh.astype(jnp.bfloat16)
        out_ref[:, r * 512:(r + 1) * 512] = jax.lax.dot_general(
            fsum_ref[...].astype(jnp.bfloat16), chunk_ref[...],
            (((1,), (0,)), ((), ())),
            preferred_element_type=f32)


@jax.jit
def kernel(x, core_W, thresholds, axon_idx, out_idx):
    B, IN = x.shape
    C, COUT, CIN = core_W.shape
    R = C * CIN
    F = C * COUT
    ONN = out_idx.shape[0]

    word = _spike_words()
    wrT = core_W.transpose(0, 2, 1).reshape(R, COUT)
    af = axon_idx.reshape(-1).astype(jnp.int32)
    adj_in = jnp.where(af < IN, af, -1).reshape(R // 512, 512)
    adj_rec4 = jnp.where(af >= IN, (af - IN) // 8, -1).reshape(R // 512, 512)
    shift_row = jnp.where(af >= IN, (af - IN) % 8, 0).reshape(1, R)
    i = jnp.arange(512)
    pk = ((i[:, None] // 8 == jnp.arange(64)[None, :]).astype(jnp.int32)
          * (1 << (i % 8))[:, None]).astype(jnp.bfloat16)
    oidx = out_idx.astype(jnp.int32).reshape(ONN // 512, 512)

    vm = pl.BlockSpec(memory_space=pltpu.VMEM)
    sm = pl.BlockSpec(memory_space=pltpu.SMEM)

    return pl.pallas_call(
        _body,
        out_shape=jax.ShapeDtypeStruct((B, ONN), jnp.float32),
        in_specs=[vm, vm, sm, vm, vm, vm, vm, vm, sm],
        out_specs=vm,
        compiler_params=pltpu.CompilerParams(
            vmem_limit_bytes=100 * 1024 * 1024),
        scratch_shapes=[
            pltpu.VMEM((2, B, IN), jnp.bfloat16),
            pltpu.VMEM((F // 8, R), jnp.bfloat16),
            pltpu.VMEM((F, 512), jnp.bfloat16),
            pltpu.VMEM((B, R), jnp.int32),
            pltpu.VMEM((B, F), jnp.float32),
            pltpu.VMEM((2, B, F // 8), jnp.bfloat16),
            pltpu.VMEM((B, F), jnp.float32),
        ],
    )(x, wrT, thresholds, adj_in, adj_rec4, shift_row, pk, oidx, word)


# speedup vs baseline: 5.8886x; 1.2134x over previous
import jax
import jax.numpy as jnp
from jax.experimental import pallas as pl
from jax.experimental.pallas import tpu as pltpu

_T = 16


def _spike_words():
    xs = jnp.arange(17, dtype=jnp.float32) / 16.0
    word = jnp.zeros((17,), dtype=jnp.int32)
    for cycle in range(_T):
        N = jnp.round(xs * _T).astype(jnp.int32)
        mask = (N != 0) & (N != _T) & (cycle < _T)
        N_safe = jnp.maximum(N, 1)
        spacing = _T / N_safe.astype(jnp.float32)
        res = (
            mask
            & (jnp.floor(cycle / spacing) < N_safe.astype(jnp.float32))
            & (jnp.floor(jnp.mod(float(cycle), spacing)) == 0)
        )
        res = res.astype(jnp.float32)
        res = jnp.where(N == _T, 1.0, res)
        word = word | (res.astype(jnp.int32) << cycle)
    return word


def _body(x_ref, wrT_ref, thr_ref, adj_in_ref, adj_rec8_ref, shift_ref,
          pk_ref, oidx_ref, word_ref, out_ref, pat_ref, precT_ref,
          gpat_ref, memb_ref, fb_ref, fsum_ref, inc_ref):
    IN = x_ref.shape[1]
    F = fsum_ref.shape[1]
    B = x_ref.shape[0]
    CW = pk_ref.shape[0]
    C = F // CW
    F8 = F // 8
    f32 = jnp.float32
    bf16 = jnp.bfloat16

    n_int = jnp.round(x_ref[...] * float(_T)).astype(jnp.int32)
    acc = jnp.zeros((B, IN), dtype=jnp.int32)
    for n in range(17):
        acc = jnp.where(n_int == n, word_ref[n], acc)
    pat_ref[0] = (acc & 255).astype(bf16)
    pat_ref[1] = (acc >> 8).astype(bf16)

    for r in range(adj_in_ref.shape[0]):
        idx = adj_in_ref[r:r + 1, :]
        oh = (jax.lax.broadcasted_iota(jnp.int32, (IN, 512), 0)
              == idx).astype(bf16)
        glo = jax.lax.dot_general(
            pat_ref[0], oh, (((1,), (0,)), ((), ())),
            preferred_element_type=f32)
        ghi = jax.lax.dot_general(
            pat_ref[1], oh, (((1,), (0,)), ((), ())),
            preferred_element_type=f32)
        gpat_ref[:, r * 512:(r + 1) * 512] = (
            (ghi.astype(jnp.int32) << 8) | glo.astype(jnp.int32))

    for r in range(adj_rec8_ref.shape[0]):
        idx = adj_rec8_ref[r:r + 1, :]
        oh = (jax.lax.broadcasted_iota(jnp.int32, (F8, 512), 0) == idx)
        precT_ref[:, r * 512:(r + 1) * 512] = oh.astype(bf16)

    memb_ref[...] = jnp.zeros((B, F), f32)
    fsum_ref[...] = jnp.zeros((B, F), f32)
    fb_ref[0] = jnp.zeros((B, F8), bf16)

    for t in range(_T):
        cur = t % 2
        git = (gpat_ref[...] >> t) & 1
        g_rec = jax.lax.dot_general(
            fb_ref[cur], precT_ref[...],
            (((1,), (0,)), ((), ())),
            preferred_element_type=f32)
        rec_bit = (g_rec.astype(jnp.int32) >> shift_ref[...]) & 1
        sig = (rec_bit + git).astype(f32)
        for c in range(C):
            cs = c * CW
            inc_ref[:, cs:cs + CW] = jax.lax.dot_general(
                sig[:, cs:cs + CW], wrT_ref[cs:cs + CW, :],
                (((1,), (0,)), ((), ())),
                preferred_element_type=f32)
        thr = thr_ref[...]
        memb = memb_ref[...] + inc_ref[...]
        fired = (thr < memb).astype(f32)
        memb_ref[...] = memb - fired * thr
        fsum_ref[...] = fsum_ref[...] + fired
        for c in range(C):
            cs = c * CW
            pf = jax.lax.dot_general(
                fired[:, cs:cs + CW].astype(bf16), pk_ref[...],
                (((1,), (0,)), ((), ())),
                preferred_element_type=f32)
            fb_ref[1 - cur, :, c * (CW // 8):(c + 1) * (CW // 8)] = (
                pf.astype(bf16))

    fsum_bf = fsum_ref[...].astype(bf16)
    for r in range(oidx_ref.shape[0]):
        idx = oidx_ref[r:r + 1, :]
        oh = (jax.lax.broadcasted_iota(jnp.int32, (F, 512), 0)
              == idx).astype(bf16)
        out_ref[:, r * 512:(r + 1) * 512] = jax.lax.dot_general(
            fsum_bf, oh, (((1,), (0,)), ((), ())),
            preferred_element_type=f32)


@jax.jit
def kernel(x, core_W, thresholds, axon_idx, out_idx):
    B, IN = x.shape
    C, COUT, CIN = core_W.shape
    R = C * CIN
    F = C * COUT
    ONN = out_idx.shape[0]

    word = _spike_words()
    wrT = core_W.transpose(0, 2, 1).reshape(R, COUT)
    thr_row = jnp.repeat(thresholds, COUT).reshape(1, F)
    af = axon_idx.reshape(-1).astype(jnp.int32)
    adj_in = jnp.where(af < IN, af, -1).reshape(R // 512, 512)
    adj_rec8 = jnp.where(af >= IN, (af - IN) // 8, -1).reshape(R // 512, 512)
    shift_row = jnp.where(af >= IN, (af - IN) % 8, 0).reshape(1, R)
    i = jnp.arange(512)
    pk = ((i[:, None] // 8 == jnp.arange(64)[None, :]).astype(jnp.int32)
          * (1 << (i % 8))[:, None]).astype(jnp.bfloat16)
    oidx = out_idx.astype(jnp.int32).reshape(ONN // 512, 512)

    vm = pl.BlockSpec(memory_space=pltpu.VMEM)
    sm = pl.BlockSpec(memory_space=pltpu.SMEM)

    return pl.pallas_call(
        _body,
        out_shape=jax.ShapeDtypeStruct((B, ONN), jnp.float32),
        in_specs=[vm, vm, vm, vm, vm, vm, vm, vm, sm],
        out_specs=vm,
        compiler_params=pltpu.CompilerParams(
            vmem_limit_bytes=100 * 1024 * 1024),
        scratch_shapes=[
            pltpu.VMEM((2, B, IN), jnp.bfloat16),
            pltpu.VMEM((F // 8, R), jnp.bfloat16),
            pltpu.VMEM((B, R), jnp.int32),
            pltpu.VMEM((B, F), jnp.float32),
            pltpu.VMEM((2, B, F // 8), jnp.bfloat16),
            pltpu.VMEM((B, F), jnp.float32),
            pltpu.VMEM((B, F), jnp.float32),
        ],
    )(x, wrT, thr_row, adj_in, adj_rec8, shift_row, pk, oidx, word)


# speedup vs baseline: 6.1305x; 1.0411x over previous
import jax
import jax.numpy as jnp
from jax.experimental import pallas as pl
from jax.experimental.pallas import tpu as pltpu

_T = 16


def _spike_words():
    xs = jnp.arange(17, dtype=jnp.float32) / 16.0
    word = jnp.zeros((17,), dtype=jnp.int32)
    for cycle in range(_T):
        N = jnp.round(xs * _T).astype(jnp.int32)
        mask = (N != 0) & (N != _T) & (cycle < _T)
        N_safe = jnp.maximum(N, 1)
        spacing = _T / N_safe.astype(jnp.float32)
        res = (
            mask
            & (jnp.floor(cycle / spacing) < N_safe.astype(jnp.float32))
            & (jnp.floor(jnp.mod(float(cycle), spacing)) == 0)
        )
        res = res.astype(jnp.float32)
        res = jnp.where(N == _T, 1.0, res)
        word = word | (res.astype(jnp.int32) << cycle)
    return word


def _body(x_ref, wrT_ref, thr_ref, adj_in_ref, adj_rec8_ref, shift_ref,
          inp_ref, pk_ref, oidx_ref, word_ref, out_ref, pat_ref,
          precT_ref, gpat_ref, memb_ref, fb_ref, fsum_ref, inc_ref):
    IN = x_ref.shape[1]
    F = fsum_ref.shape[1]
    B = x_ref.shape[0]
    CW = pk_ref.shape[0]
    C = F // CW
    F8 = F // 8
    f32 = jnp.float32
    bf16 = jnp.bfloat16

    n_int = jnp.round(x_ref[...] * float(_T)).astype(jnp.int32)
    acc = jnp.zeros((B, IN), dtype=jnp.int32)
    for n in range(17):
        acc = jnp.where(n_int == n, word_ref[n], acc)
    pat_ref[:B, :] = (acc & 255).astype(bf16)
    pat_ref[B:, :] = (acc >> 8).astype(bf16)

    for r in range(adj_in_ref.shape[0]):
        idx = adj_in_ref[r:r + 1, :]
        oh = (jax.lax.broadcasted_iota(jnp.int32, (IN, 512), 0)
              == idx).astype(bf16)
        g = jax.lax.dot_general(
            pat_ref[...], oh, (((1,), (0,)), ((), ())),
            preferred_element_type=f32)
        gpat_ref[:, r * 512:(r + 1) * 512] = (
            (g[B:, :].astype(jnp.int32) << 8) | g[:B, :].astype(jnp.int32))

    for r in range(adj_rec8_ref.shape[0]):
        idx = adj_rec8_ref[r:r + 1, :]
        oh = (jax.lax.broadcasted_iota(jnp.int32, (F8, 512), 0) == idx)
        precT_ref[:, r * 512:(r + 1) * 512] = oh.astype(bf16)

    memb_ref[...] = jnp.zeros((B, F), f32)
    fsum_ref[...] = jnp.zeros((B, F), f32)
    fb_ref[0] = jnp.zeros((B, F8), bf16)

    for t in range(_T):
        cur = t % 2
        g_rec = jax.lax.dot_general(
            fb_ref[cur], precT_ref[...],
            (((1,), (0,)), ((), ())),
            preferred_element_type=f32)
        u = g_rec.astype(jnp.int32) | gpat_ref[...]
        sh = shift_ref[...] + t * inp_ref[...]
        sig = ((u >> sh) & 1).astype(f32)
        for c in range(C):
            cs = c * CW
            inc_ref[:, cs:cs + CW] = jax.lax.dot_general(
                sig[:, cs:cs + CW], wrT_ref[cs:cs + CW, :],
                (((1,), (0,)), ((), ())),
                preferred_element_type=f32)
        thr = thr_ref[...]
        memb = memb_ref[...] + inc_ref[...]
        fired = (thr < memb).astype(f32)
        memb_ref[...] = memb - fired * thr
        fsum_ref[...] = fsum_ref[...] + fired
        for c in range(C):
            cs = c * CW
            pf = jax.lax.dot_general(
                fired[:, cs:cs + CW].astype(bf16), pk_ref[...],
                (((1,), (0,)), ((), ())),
                preferred_element_type=f32)
            fb_ref[1 - cur, :, c * (CW // 8):(c + 1) * (CW // 8)] = (
                pf.astype(bf16))

    fsum_bf = fsum_ref[...].astype(bf16)
    for r in range(oidx_ref.shape[0]):
        idx = oidx_ref[r:r + 1, :]
        oh = (jax.lax.broadcasted_iota(jnp.int32, (F, 512), 0)
              == idx).astype(bf16)
        out_ref[:, r * 512:(r + 1) * 512] = jax.lax.dot_general(
            fsum_bf, oh, (((1,), (0,)), ((), ())),
            preferred_element_type=f32)


@jax.jit
def kernel(x, core_W, thresholds, axon_idx, out_idx):
    B, IN = x.shape
    C, COUT, CIN = core_W.shape
    R = C * CIN
    F = C * COUT
    ONN = out_idx.shape[0]

    word = _spike_words()
    wrT = core_W.transpose(0, 2, 1).reshape(R, COUT)
    thr_row = jnp.repeat(thresholds, COUT).reshape(1, F)
    af = axon_idx.reshape(-1).astype(jnp.int32)
    adj_in = jnp.where(af < IN, af, -1).reshape(R // 512, 512)
    adj_rec8 = jnp.where(af >= IN, (af - IN) // 8, -1).reshape(R // 512, 512)
    shift_row = jnp.where(af >= IN, (af - IN) % 8, 0).reshape(1, R)
    inp_row = (af < IN).astype(jnp.int32).reshape(1, R)
    i = jnp.arange(512)
    pk = ((i[:, None] // 8 == jnp.arange(64)[None, :]).astype(jnp.int32)
          * (1 << (i % 8))[:, None]).astype(jnp.bfloat16)
    oidx = out_idx.astype(jnp.int32).reshape(ONN // 512, 512)

    vm = pl.BlockSpec(memory_space=pltpu.VMEM)
    sm = pl.BlockSpec(memory_space=pltpu.SMEM)

    return pl.pallas_call(
        _body,
        out_shape=jax.ShapeDtypeStruct((B, ONN), jnp.float32),
        in_specs=[vm, vm, vm, vm, vm, vm, vm, vm, vm, sm],
        out_specs=vm,
        compiler_params=pltpu.CompilerParams(
            vmem_limit_bytes=100 * 1024 * 1024),
        scratch_shapes=[
            pltpu.VMEM((2 * B, IN), jnp.bfloat16),
            pltpu.VMEM((F // 8, R), jnp.bfloat16),
            pltpu.VMEM((B, R), jnp.int32),
            pltpu.VMEM((B, F), jnp.float32),
            pltpu.VMEM((2, B, F // 8), jnp.bfloat16),
            pltpu.VMEM((B, F), jnp.float32),
            pltpu.VMEM((B, F), jnp.float32),
        ],
    )(x, wrT, thr_row, adj_in, adj_rec8, shift_row, inp_row, pk, oidx, word)
